# baseline (speedup 1.0000x reference)
import functools

import jax
import jax.numpy as jnp
from jax import lax
from jax.experimental import pallas as pl
from jax.experimental.pallas import tpu as pltpu
from jax.experimental.pallas import tpu_sc as plsc

NC, NS = 2, 16
NW = NC * NS
C = 80


@functools.lru_cache(maxsize=None)
def _build(B, D):
    BPW = B // NW
    NCH = BPW // C
    assert BPW * NW == B and NCH * C == BPW and NCH % 2 == 0

    mesh = plsc.VectorSubcoreMesh(core_axis_name="c", subcore_axis_name="s")

    @functools.partial(
        pl.kernel,
        out_type=jax.ShapeDtypeStruct((B, D), jnp.float32),
        mesh=mesh,
        scratch_types=[
            pltpu.VMEM((BPW,), jnp.int32),
            pltpu.VMEM((C, D), jnp.float32),
            pltpu.VMEM((C, D), jnp.float32),
            pltpu.SemaphoreType.DMA,
            pltpu.SemaphoreType.DMA,
            pltpu.SemaphoreType.DMA,
            pltpu.SemaphoreType.DMA,
        ],
    )
    def emb(idx_hbm, table_hbm, out_hbm, idx_v, rows0, rows1, g0, g1, o0, o1):
        rows = (rows0, rows1)
        gsem = (g0, g1)
        osem = (o0, o1)
        wid = lax.axis_index("s") * NC + lax.axis_index("c")
        base = wid * BPW
        pltpu.sync_copy(idx_hbm.at[pl.ds(base, BPW)], idx_v)

        def start_g(c, b):
            pltpu.async_copy(table_hbm.at[idx_v.at[pl.ds(c * C, C)]],
                             rows[b], gsem[b])

        def wait_g(b):
            pltpu.make_async_copy(table_hbm.at[idx_v.at[pl.ds(0, C)]],
                                  rows[b], gsem[b]).wait()

        def start_o(c, b):
            pltpu.async_copy(rows[b], out_hbm.at[pl.ds(base + c * C, C)],
                             osem[b])

        def wait_o(b):
            pltpu.make_async_copy(rows[b], out_hbm.at[pl.ds(0, C)],
                                  osem[b]).wait()

        start_g(0, 0)

        def step(c, b):
            wait_g(b)
            start_o(c, b)

            @pl.when(c + 1 < NCH)
            def _():
                @pl.when(c >= 1)
                def _():
                    wait_o(1 - b)
                start_g(c + 1, 1 - b)

        def body(i, carry):
            step(2 * i, 0)
            step(2 * i + 1, 1)
            return carry

        lax.fori_loop(0, NCH // 2, body, 0)
        wait_o(0)
        wait_o(1)

    return emb


def kernel(word_seq, embedding_table):
    s, t = word_seq.shape
    b = s * t
    idx = word_seq.reshape(b).astype(jnp.int32)
    table = embedding_table.astype(jnp.float32)
    out = _build(b, embedding_table.shape[1])(idx, table)
    return out.reshape(s, t, embedding_table.shape[1])
